# baseline (speedup 1.0000x reference)
import functools

import jax
import jax.numpy as jnp
from jax import lax
from jax.experimental import pallas as pl
from jax.experimental.pallas import tpu as pltpu
from jax.experimental.pallas import tpu_sc as plsc

NUM_CLASS = 1000
MEM_SIZE = 64
D = 512
B = 16384
ROWS = NUM_CLASS * MEM_SIZE

_BB = 512
_SROWS = B // _BB

_NC = 2
_NS = 16
_NW = _NC * _NS
_EPT = B // _NS
_VPT = _EPT // 16
_RPW = ROWS // _NW
_CHUNK = 40
_NCHUNK = _RPW // _CHUNK
_DUMMY = ROWS
_STRIPE = 4016
_TSH = _NS * _STRIPE


def _prep_body(logits_ref, slot_ref, flat_ref):
    lg = logits_ref[...]
    m = jnp.max(lg, axis=0, keepdims=True)
    row = lax.broadcasted_iota(jnp.int32, lg.shape, 0)
    idx = jnp.min(jnp.where(lg == m, row, NUM_CLASS), axis=0)
    flat_ref[...] = (idx * MEM_SIZE + slot_ref[0, 0]).reshape(1, 1, _BB)


def _sc_body(flat_hbm, feat_hbm, mem_hbm, out_hbm,
             flat8, ibuf, sfbuf, wloc, neg1, tbuf, rb0, rb1, tsh,
             sem_g0, sem_g1, sem_w0, sem_w1, sem_f):
    c = lax.axis_index("c")
    s = lax.axis_index("s")
    w = c * _NS + s
    lane = lax.iota(jnp.int32, 16)
    shift = jnp.minimum(lane + 1, 15)

    pltpu.sync_copy(flat_hbm.at[pl.ds(s * 8, 8)], flat8)

    def init_body(j, _):
        neg1[pl.ds(j * 16, 16)] = jnp.full((16,), -1, jnp.int32)
        return 0
    lax.fori_loop(0, _STRIPE // 16, init_body, 0)
    pltpu.sync_copy(neg1, tsh.at[pl.ds(s * _STRIPE, _STRIPE)])

    def dedup_body(j, _):
        f = flat8[j // 8, pl.ds((j % 8) * 16, 16)]
        i = s * _EPT + j * 16 + lane
        key = f * 16384 + i
        ks, _ = plsc.sort_key_val(key, key)
        fs = lax.shift_right_arithmetic(ks, 14)
        isort = lax.bitwise_and(ks, 16383)
        nxt = lax.gather(
            fs, shift.reshape(16, 1),
            lax.GatherDimensionNumbers(offset_dims=(), collapsed_slice_dims=(0,),
                                       start_index_map=(0,)),
            slice_sizes=(1,), mode=lax.GatherScatterMode.PROMISE_IN_BOUNDS)
        lastm = jnp.logical_or(nxt != fs, lane == 15)
        plsc.store_scatter(wloc, [fs], isort, mask=lastm)
        return 0
    lax.fori_loop(0, _VPT, dedup_body, 0)

    def live_body(j, _):
        f = flat8[j // 8, pl.ds((j % 8) * 16, 16)]
        i = s * _EPT + j * 16 + lane
        wv = plsc.load_gather(wloc, [f])
        live = wv == i
        sfbuf[j // 8, pl.ds((j % 8) * 16, 16)] = jnp.where(live, f, _DUMMY)
        ibuf[j // 8, pl.ds((j % 8) * 16, 16)] = i
        return 0
    lax.fori_loop(0, _VPT, live_body, 0)

    plsc.subcore_barrier()

    def round_body(r, _):
        @pl.when(s == r)
        def _():
            for r8 in range(8):
                pltpu.async_copy(ibuf.at[r8], tsh.at[sfbuf.at[r8]], sem_f)
            for r8 in range(8):
                pltpu.make_async_copy(ibuf.at[r8], tsh.at[sfbuf.at[r8]],
                                      sem_f).wait()
        plsc.subcore_barrier()
        return 0
    lax.fori_loop(0, _NS, round_body, 0)

    base = w * _RPW
    pltpu.sync_copy(tsh.at[pl.ds(base, _RPW)], tbuf)

    def gather_issue(k, rb, sem):
        pltpu.async_copy(mem_hbm.at[pl.ds(base + k * _CHUNK, _CHUNK)],
                         rb, sem)

    def group_fix(k, goff, valid_lo, rb):
        tv = tbuf[pl.ds(k * _CHUNK + goff, 16)]
        has0 = jnp.logical_and(tv >= 0, lane >= valid_lo)

        def fcond(st):
            has, _ = st
            return jnp.any(has)

        def fbody(st):
            has, cnt = st
            l = jnp.max(plsc.all_reduce_ffs(has))
            g = jnp.max(jnp.where(lane == l, tv, -1))
            pltpu.async_copy(feat_hbm.at[pl.ds(g, 1)],
                             rb.at[pl.ds(goff + l, 1)], sem_f)
            return jnp.logical_and(has, lane != l), cnt + 1

        _, cnt = lax.while_loop(fcond, fbody, (has0, jnp.int32(0)))
        return cnt

    def norm_row(rb, r):
        def accum(k2, acc):
            v = rb[r, pl.ds(k2 * 16, 16)]
            return acc + v * v
        acc = lax.fori_loop(0, D // 16, accum, jnp.zeros((16,), jnp.float32))
        sv = jnp.broadcast_to(jnp.sum(acc), (16,))
        yi = 0x5F3759DF - lax.shift_right_logical(plsc.bitcast(sv, jnp.int32), 1)
        y = plsc.bitcast(yi, jnp.float32)
        for _ in range(3):
            y = y * (1.5 - 0.5 * sv * y * y)

        def scale(k2, _):
            rb[r, pl.ds(k2 * 16, 16)] = rb[r, pl.ds(k2 * 16, 16)] * y
            return 0
        lax.fori_loop(0, D // 16, scale, 0)

    def group_norm(k, goff, valid_lo, rb):
        tv = tbuf[pl.ds(k * _CHUNK + goff, 16)]
        has0 = jnp.logical_and(tv >= 0, lane >= valid_lo)

        def ncond(has):
            return jnp.any(has)

        def nbody(has):
            l = jnp.max(plsc.all_reduce_ffs(has))
            norm_row(rb, goff + l)
            return jnp.logical_and(has, lane != l)

        lax.while_loop(ncond, nbody, has0)

    def process_chunk(k, rb, sem_g, sem_w):
        pltpu.make_async_copy(mem_hbm.at[pl.ds(0, _CHUNK)], rb, sem_g).wait()
        cnt = group_fix(k, 0, 0, rb)
        cnt = cnt + group_fix(k, 16, 0, rb)
        cnt = cnt + group_fix(k, 24, 8, rb)

        def drain_body(t, _):
            pltpu.make_async_copy(feat_hbm.at[pl.ds(0, 1)],
                                  rb.at[pl.ds(0, 1)], sem_f).wait()
            return 0
        lax.fori_loop(0, cnt, drain_body, 0)

        @pl.when(cnt > 0)
        def _():
            group_norm(k, 0, 0, rb)
            group_norm(k, 16, 0, rb)
            group_norm(k, 24, 8, rb)
        pltpu.async_copy(rb, out_hbm.at[pl.ds(base + k * _CHUNK, _CHUNK)],
                         sem_w)

        @pl.when(k + 2 < _NCHUNK)
        def _():
            pltpu.make_async_copy(rb, out_hbm.at[pl.ds(base, _CHUNK)],
                                  sem_w).wait()
            gather_issue(k + 2, rb, sem_g)

    gather_issue(0, rb0, sem_g0)
    gather_issue(1, rb1, sem_g1)

    def pair_body(kk, _):
        process_chunk(kk * 2, rb0, sem_g0, sem_w0)
        process_chunk(kk * 2 + 1, rb1, sem_g1, sem_w1)
        return 0
    lax.fori_loop(0, _NCHUNK // 2, pair_body, 0)
    pltpu.make_async_copy(rb0, out_hbm.at[pl.ds(base, _CHUNK)], sem_w0).wait()
    pltpu.make_async_copy(rb1, out_hbm.at[pl.ds(base, _CHUNK)], sem_w1).wait()


@jax.jit
def kernel(image_feature_memory, logits, image_features, slot_idx):
    slot3d = slot_idx.reshape(_SROWS, 1, _BB)
    flat3d = pl.pallas_call(
        _prep_body,
        grid=(_SROWS,),
        in_specs=[
            pl.BlockSpec((NUM_CLASS, _BB), lambda i: (0, i)),
            pl.BlockSpec((1, 1, _BB), lambda i: (i, 0, 0)),
        ],
        out_specs=pl.BlockSpec((1, 1, _BB), lambda i: (i, 0, 0)),
        out_shape=jax.ShapeDtypeStruct((_SROWS, 1, _BB), jnp.int32),
    )(logits.T, slot3d)
    flat2d = flat3d.reshape(B // 128, 128)

    sc = pl.kernel(
        _sc_body,
        out_type=jax.ShapeDtypeStruct((ROWS, D), jnp.float32),
        mesh=plsc.VectorSubcoreMesh(core_axis_name="c", subcore_axis_name="s"),
        compiler_params=pltpu.CompilerParams(needs_layout_passes=False),
        scratch_types=[
            pltpu.VMEM((8, 128), jnp.int32),
            pltpu.VMEM((8, 128), jnp.int32),
            pltpu.VMEM((8, 128), jnp.int32),
            pltpu.VMEM((ROWS,), jnp.int32),
            pltpu.VMEM((_STRIPE,), jnp.int32),
            pltpu.VMEM((_RPW,), jnp.int32),
            pltpu.VMEM((_CHUNK, D), jnp.float32),
            pltpu.VMEM((_CHUNK, D), jnp.float32),
            pltpu.VMEM_SHARED((_TSH,), jnp.int32),
            pltpu.SemaphoreType.DMA,
            pltpu.SemaphoreType.DMA,
            pltpu.SemaphoreType.DMA,
            pltpu.SemaphoreType.DMA,
            pltpu.SemaphoreType.DMA,
        ],
    )
    return sc(flat2d, image_features, image_feature_memory)


# speedup vs baseline: 1.7869x; 1.7869x over previous
import functools

import jax
import jax.numpy as jnp
from jax import lax
from jax.experimental import pallas as pl
from jax.experimental.pallas import tpu as pltpu
from jax.experimental.pallas import tpu_sc as plsc

NUM_CLASS = 1000
MEM_SIZE = 64
D = 512
B = 16384
ROWS = NUM_CLASS * MEM_SIZE

_BB = 512
_SROWS = B // _BB

_NC = 2
_NS = 16
_NW = _NC * _NS
_EPT = B // _NS
_VPT = _EPT // 16
_RPW = ROWS // _NW
_CHUNK = 40
_NCHUNK = _RPW // _CHUNK
_DUMMY = ROWS
_STRIPE = 4016
_TSH = _NS * _STRIPE


def _prep_body(logits_ref, feat_ref, slot_ref, flat_ref, fnorm_ref):
    lg = logits_ref[...]
    m = jnp.max(lg, axis=0, keepdims=True)
    row = lax.broadcasted_iota(jnp.int32, lg.shape, 0)
    idx = jnp.min(jnp.where(lg == m, row, NUM_CLASS), axis=0)
    flat_ref[...] = (idx * MEM_SIZE + slot_ref[0, 0]).reshape(1, 1, _BB)
    x = feat_ref[...]
    s = jnp.sum(x * x, axis=1, keepdims=True)
    fnorm_ref[...] = x * lax.rsqrt(s)


def _sc_body(flat_hbm, feat_hbm, mem_hbm, out_hbm,
             flat8, ibuf, sfbuf, wloc, neg1, tbuf, rb0, rb1, tsh,
             sem_g0, sem_g1, sem_w0, sem_w1, sem_f):
    c = lax.axis_index("c")
    s = lax.axis_index("s")
    w = c * _NS + s
    lane = lax.iota(jnp.int32, 16)
    shift = jnp.minimum(lane + 1, 15)

    pltpu.sync_copy(flat_hbm.at[pl.ds(s * 8, 8)], flat8)

    def init_body(j, _):
        neg1[pl.ds(j * 16, 16)] = jnp.full((16,), -1, jnp.int32)
        return 0
    lax.fori_loop(0, _STRIPE // 16, init_body, 0)
    pltpu.sync_copy(neg1, tsh.at[pl.ds(s * _STRIPE, _STRIPE)])

    def dedup_body(j, _):
        f = flat8[j // 8, pl.ds((j % 8) * 16, 16)]
        i = s * _EPT + j * 16 + lane
        key = f * 16384 + i
        ks, _ = plsc.sort_key_val(key, key)
        fs = lax.shift_right_arithmetic(ks, 14)
        isort = lax.bitwise_and(ks, 16383)
        nxt = lax.gather(
            fs, shift.reshape(16, 1),
            lax.GatherDimensionNumbers(offset_dims=(), collapsed_slice_dims=(0,),
                                       start_index_map=(0,)),
            slice_sizes=(1,), mode=lax.GatherScatterMode.PROMISE_IN_BOUNDS)
        lastm = jnp.logical_or(nxt != fs, lane == 15)
        plsc.store_scatter(wloc, [fs], isort, mask=lastm)
        return 0
    lax.fori_loop(0, _VPT, dedup_body, 0)

    def live_body(j, _):
        f = flat8[j // 8, pl.ds((j % 8) * 16, 16)]
        i = s * _EPT + j * 16 + lane
        wv = plsc.load_gather(wloc, [f])
        live = wv == i
        sfbuf[j // 8, pl.ds((j % 8) * 16, 16)] = jnp.where(live, f, _DUMMY)
        ibuf[j // 8, pl.ds((j % 8) * 16, 16)] = i
        return 0
    lax.fori_loop(0, _VPT, live_body, 0)

    plsc.subcore_barrier()

    def round_body(r, _):
        @pl.when(s == r)
        def _():
            for r8 in range(8):
                pltpu.async_copy(ibuf.at[r8], tsh.at[sfbuf.at[r8]], sem_f)
            for r8 in range(8):
                pltpu.make_async_copy(ibuf.at[r8], tsh.at[sfbuf.at[r8]],
                                      sem_f).wait()
        plsc.subcore_barrier()
        return 0
    lax.fori_loop(0, _NS, round_body, 0)

    base = w * _RPW
    pltpu.sync_copy(tsh.at[pl.ds(base, _RPW)], tbuf)

    def gather_issue(k, rb, sem):
        pltpu.async_copy(mem_hbm.at[pl.ds(base + k * _CHUNK, _CHUNK)],
                         rb, sem)

    def group_fix(k, goff, valid_lo, rb):
        tv = tbuf[pl.ds(k * _CHUNK + goff, 16)]
        has0 = jnp.logical_and(tv >= 0, lane >= valid_lo)

        def fcond(st):
            has, _ = st
            return jnp.any(has)

        def fbody(st):
            has, cnt = st
            l = jnp.max(plsc.all_reduce_ffs(has))
            g = jnp.max(jnp.where(lane == l, tv, -1))
            pltpu.async_copy(feat_hbm.at[pl.ds(g, 1)],
                             rb.at[pl.ds(goff + l, 1)], sem_f)
            return jnp.logical_and(has, lane != l), cnt + 1

        _, cnt = lax.while_loop(fcond, fbody, (has0, jnp.int32(0)))
        return cnt

    def process_chunk(k, rb, sem_g, sem_w):
        pltpu.make_async_copy(mem_hbm.at[pl.ds(0, _CHUNK)], rb, sem_g).wait()
        cnt = group_fix(k, 0, 0, rb)
        cnt = cnt + group_fix(k, 16, 0, rb)
        cnt = cnt + group_fix(k, 24, 8, rb)

        def drain_body(t, _):
            pltpu.make_async_copy(feat_hbm.at[pl.ds(0, 1)],
                                  rb.at[pl.ds(0, 1)], sem_f).wait()
            return 0
        lax.fori_loop(0, cnt, drain_body, 0)
        pltpu.async_copy(rb, out_hbm.at[pl.ds(base + k * _CHUNK, _CHUNK)],
                         sem_w)

        @pl.when(k + 2 < _NCHUNK)
        def _():
            pltpu.make_async_copy(rb, out_hbm.at[pl.ds(base, _CHUNK)],
                                  sem_w).wait()
            gather_issue(k + 2, rb, sem_g)

    gather_issue(0, rb0, sem_g0)
    gather_issue(1, rb1, sem_g1)

    def pair_body(kk, _):
        process_chunk(kk * 2, rb0, sem_g0, sem_w0)
        process_chunk(kk * 2 + 1, rb1, sem_g1, sem_w1)
        return 0
    lax.fori_loop(0, _NCHUNK // 2, pair_body, 0)
    pltpu.make_async_copy(rb0, out_hbm.at[pl.ds(base, _CHUNK)], sem_w0).wait()
    pltpu.make_async_copy(rb1, out_hbm.at[pl.ds(base, _CHUNK)], sem_w1).wait()


@jax.jit
def kernel(image_feature_memory, logits, image_features, slot_idx):
    slot3d = slot_idx.reshape(_SROWS, 1, _BB)
    flat3d, fnorm = pl.pallas_call(
        _prep_body,
        grid=(_SROWS,),
        in_specs=[
            pl.BlockSpec((NUM_CLASS, _BB), lambda i: (0, i)),
            pl.BlockSpec((_BB, D), lambda i: (i, 0)),
            pl.BlockSpec((1, 1, _BB), lambda i: (i, 0, 0)),
        ],
        out_specs=[
            pl.BlockSpec((1, 1, _BB), lambda i: (i, 0, 0)),
            pl.BlockSpec((_BB, D), lambda i: (i, 0)),
        ],
        out_shape=[
            jax.ShapeDtypeStruct((_SROWS, 1, _BB), jnp.int32),
            jax.ShapeDtypeStruct((B, D), jnp.float32),
        ],
    )(logits.T, image_features, slot3d)
    flat2d = flat3d.reshape(B // 128, 128)

    sc = pl.kernel(
        _sc_body,
        out_type=jax.ShapeDtypeStruct((ROWS, D), jnp.float32),
        mesh=plsc.VectorSubcoreMesh(core_axis_name="c", subcore_axis_name="s"),
        compiler_params=pltpu.CompilerParams(needs_layout_passes=False),
        scratch_types=[
            pltpu.VMEM((8, 128), jnp.int32),
            pltpu.VMEM((8, 128), jnp.int32),
            pltpu.VMEM((8, 128), jnp.int32),
            pltpu.VMEM((ROWS,), jnp.int32),
            pltpu.VMEM((_STRIPE,), jnp.int32),
            pltpu.VMEM((_RPW,), jnp.int32),
            pltpu.VMEM((_CHUNK, D), jnp.float32),
            pltpu.VMEM((_CHUNK, D), jnp.float32),
            pltpu.VMEM_SHARED((_TSH,), jnp.int32),
            pltpu.SemaphoreType.DMA,
            pltpu.SemaphoreType.DMA,
            pltpu.SemaphoreType.DMA,
            pltpu.SemaphoreType.DMA,
            pltpu.SemaphoreType.DMA,
        ],
    )
    return sc(flat2d, fnorm, image_feature_memory)
